# baseline (speedup 1.0000x reference)
import functools

import jax
import jax.numpy as jnp
from jax import lax
from jax.experimental import pallas as pl
from jax.experimental.pallas import tpu as pltpu
from jax.experimental.pallas import tpu_sc as plsc

_B, _N, _M, _C = 16, 4096, 2048, 128


_G = 128
_S = 128
_NS = _N // _S


def _fps_body(x_ref, y_ref, z_ref, idx_ref, sx_ref, sy_ref, sz_ref, dist_ref):
    lane_g = lax.broadcasted_iota(jnp.int32, (_B, _G), 1)
    lane_s = lax.broadcasted_iota(jnp.int32, (_B, _S), 1)
    dist_ref[...] = jnp.full((_B, _N), 1e10, jnp.float32)

    def inner(j, st):
        f, cx, cy, cz, ia, xa, ya, za = st
        mj = lane_g == j
        ia = jnp.where(mj, f, ia)
        xa = jnp.where(mj, cx, xa)
        ya = jnp.where(mj, cy, ya)
        za = jnp.where(mj, cz, za)
        rv = jnp.full((_B, _S), -1.0, jnp.float32)
        rg = jnp.zeros((_B, _S), jnp.int32)
        rx = jnp.zeros((_B, _S), jnp.float32)
        ry = jnp.zeros((_B, _S), jnp.float32)
        rz = jnp.zeros((_B, _S), jnp.float32)
        for k in range(_NS):
            sl = pl.ds(k * _S, _S)
            xk = x_ref[:, sl]
            yk = y_ref[:, sl]
            zk = z_ref[:, sl]
            dxk = xk - cx
            dyk = yk - cy
            dzk = zk - cz
            dk = (dxk * dxk + dzk * dzk) + dyk * dyk
            ndk = jnp.minimum(dist_ref[:, sl], dk)
            dist_ref[:, sl] = ndk
            take = ndk > rv
            rv = jnp.maximum(rv, ndk)
            rg = jnp.where(take, lane_s + (k * _S), rg)
            rx = jnp.where(take, xk, rx)
            ry = jnp.where(take, yk, ry)
            rz = jnp.where(take, zk, rz)
        m = jnp.max(rv, axis=1, keepdims=True)
        g = jnp.min(jnp.where(rv == m, rg, _N), axis=1, keepdims=True)
        ohl = rg == g
        ncx = jnp.sum(jnp.where(ohl, rx, 0.0), axis=1, keepdims=True)
        ncy = jnp.sum(jnp.where(ohl, ry, 0.0), axis=1, keepdims=True)
        ncz = jnp.sum(jnp.where(ohl, rz, 0.0), axis=1, keepdims=True)
        return (g, ncx, ncy, ncz, ia, xa, ya, za)

    def outer(gi, st):
        f, cx, cy, cz = st
        zi = jnp.zeros((_B, _G), jnp.int32)
        zf = jnp.zeros((_B, _G), jnp.float32)
        f, cx, cy, cz, ia, xa, ya, za = lax.fori_loop(
            0, _G, inner, (f, cx, cy, cz, zi, zf, zf, zf))
        base = pl.multiple_of(gi * _G, _G)
        idx_ref[:, pl.ds(base, _G)] = ia
        sx_ref[:, pl.ds(base, _G)] = xa
        sy_ref[:, pl.ds(base, _G)] = ya
        sz_ref[:, pl.ds(base, _G)] = za
        return (f, cx, cy, cz)

    lax.fori_loop(
        0, _M // _G, outer,
        (jnp.zeros((_B, 1), jnp.int32),
         x_ref[:, 0:1], y_ref[:, 0:1], z_ref[:, 0:1]))


def _fps(x, y, z):
    return pl.pallas_call(
        _fps_body,
        out_shape=(
            jax.ShapeDtypeStruct((_B, _M), jnp.int32),
            jax.ShapeDtypeStruct((_B, _M), jnp.float32),
            jax.ShapeDtypeStruct((_B, _M), jnp.float32),
            jax.ShapeDtypeStruct((_B, _M), jnp.float32),
        ),
        scratch_shapes=[pltpu.VMEM((_B, _N), jnp.float32)],
    )(x, y, z)


_NW = 32
_CW = _C // _NW


def _gather_body(feats_hbm, idx_hbm, out_hbm, idx_v, feat_v, out_v):
    wid = lax.axis_index("s") * 2 + lax.axis_index("c")
    c0 = wid * _CW
    for b in range(_B):
        pltpu.sync_copy(idx_hbm.at[b], idx_v)
        pltpu.sync_copy(feats_hbm.at[b, pl.ds(c0, _CW)], feat_v)
        for c in range(_CW):
            cvec = jnp.full((16,), c, jnp.int32)

            def inner(jj, carry):
                for u in range(4):
                    off = jj * 64 + u * 16
                    iv = idx_v[pl.ds(off, 16)]
                    out_v[c, pl.ds(off, 16)] = plsc.load_gather(
                        feat_v, [cvec, iv])
                return carry

            lax.fori_loop(0, _M // 64, inner, 0)
        pltpu.sync_copy(out_v, out_hbm.at[b, pl.ds(c0, _CW)])


def _gather(feats, idx):
    mesh = plsc.VectorSubcoreMesh(core_axis_name="c", subcore_axis_name="s")
    return pl.kernel(
        _gather_body,
        out_type=jax.ShapeDtypeStruct((_B, _C, _M), jnp.float32),
        mesh=mesh,
        compiler_params=pltpu.CompilerParams(needs_layout_passes=False),
        scratch_types=[
            pltpu.VMEM((_M,), jnp.int32),
            pltpu.VMEM((_CW, _N), jnp.float32),
            pltpu.VMEM((_CW, _M), jnp.float32),
        ],
    )(feats, idx)


def kernel(feats, xyz):
    xt = jnp.transpose(xyz, (2, 0, 1))
    idx, sx, sy, sz = _fps(xt[0], xt[1], xt[2])
    new_xyz = jnp.stack([sx, sy, sz], axis=-1)
    new_feats = _gather(feats, idx)
    return (new_feats, new_xyz)
